# baseline (speedup 1.0000x reference)
import functools

import jax
import jax.numpy as jnp
from jax import lax
from jax.experimental import pallas as pl
from jax.experimental.pallas import tpu as pltpu
from jax.experimental.pallas import tpu_sc as plsc

N_NODES = 10000
N_EDGES = 320000
D = 128
G = 128
NG = N_EDGES // G
NC = 2
NS = 16
GROUPS_PER_CORE = NG // NC
GROUPS_PER_SUB = -(-GROUPS_PER_CORE // NS)
NPAD = 10240
ROWS_PER_SUB = NPAD // NS


def _sc_segment_sum(nbr_feat, idx1d):
    mesh = plsc.VectorSubcoreMesh(core_axis_name="c", subcore_axis_name="s")

    @functools.partial(
        pl.kernel,
        out_type=(
            jax.ShapeDtypeStruct((NC, NPAD, D), jnp.float32),
            jax.ShapeDtypeStruct((NC, NPAD), jnp.float32),
        ),
        mesh=mesh,
        scratch_types=[
            pltpu.VMEM((2, G), jnp.int32),
            pltpu.VMEM((2, G, D), jnp.float32),
            pltpu.VMEM((ROWS_PER_SUB,), jnp.float32),
            pltpu.VMEM((G,), jnp.float32),
            pltpu.VMEM_SHARED((NPAD, D), jnp.float32),
            pltpu.VMEM_SHARED((NPAD,), jnp.float32),
            pltpu.SemaphoreType.DMA,
            pltpu.SemaphoreType.DMA,
            pltpu.SemaphoreType.DMA,
            pltpu.SemaphoreType.DMA,
        ],
    )
    def k(nbr_hbm, idx_hbm, psum_hbm, pcnt_hbm,
          idx_v, row_v, zc_v, ones_v, acc_sh, cnt_sh, sem0, sem1,
          rsem, csem):
        c = lax.axis_index("c")
        s = lax.axis_index("s")
        zeros16 = jnp.zeros((16,), jnp.float32)
        ones16 = jnp.full((16,), 1.0, jnp.float32)
        sems = (sem0, sem1)

        def z_row(r, carry):
            for kk in range(D // 16):
                row_v[0, r, pl.ds(kk * 16, 16)] = zeros16
            return carry
        lax.fori_loop(0, G, z_row, None)

        def z_cnt(i, carry):
            zc_v[pl.ds(i * 16, 16)] = zeros16
            return carry
        lax.fori_loop(0, ROWS_PER_SUB // 16, z_cnt, None)

        for kk in range(G // 16):
            ones_v[pl.ds(kk * 16, 16)] = ones16

        base = s * ROWS_PER_SUB
        for j in range(ROWS_PER_SUB // G):
            pltpu.sync_copy(row_v.at[0], acc_sh.at[pl.ds(base + j * G, G)])
        pltpu.sync_copy(zc_v, cnt_sh.at[pl.ds(base, ROWS_PER_SUB)])
        plsc.subcore_barrier()

        def pred(t):
            return (t < GROUPS_PER_SUB) & (s * GROUPS_PER_SUB + t < GROUPS_PER_CORE)

        def start(t, b):
            @pl.when(pred(t))
            def _():
                gid = c * GROUPS_PER_CORE + s * GROUPS_PER_SUB + t
                pltpu.async_copy(idx_hbm.at[pl.ds(gid * G, G)], idx_v.at[b],
                                 sems[b])
                pltpu.async_copy(nbr_hbm.at[pl.ds(gid * G, G)], row_v.at[b],
                                 sems[b])

        def proc(t, b):
            @pl.when(pred(t))
            def _():
                pltpu.make_async_copy(idx_hbm.at[pl.ds(0, G)], idx_v.at[b],
                                      sems[b]).wait()
                pltpu.make_async_copy(nbr_hbm.at[pl.ds(0, G)], row_v.at[b],
                                      sems[b]).wait()
                d1 = pltpu.async_copy(row_v.at[b], acc_sh.at[idx_v.at[b]],
                                      rsem, add=True)
                d2 = pltpu.async_copy(ones_v, cnt_sh.at[idx_v.at[b]],
                                      csem, add=True)
                d1.wait()
                d2.wait()

        start(0, 0)

        def pair(p, carry):
            t0 = 2 * p
            start(t0 + 1, 1)
            proc(t0, 0)
            start(t0 + 2, 0)
            proc(t0 + 1, 1)
            return carry
        lax.fori_loop(0, (GROUPS_PER_SUB + 1) // 2, pair, None)
        plsc.subcore_barrier()

        pltpu.sync_copy(acc_sh.at[pl.ds(base, ROWS_PER_SUB)],
                        psum_hbm.at[c, pl.ds(base, ROWS_PER_SUB)])
        pltpu.sync_copy(cnt_sh.at[pl.ds(base, ROWS_PER_SUB)],
                        pcnt_hbm.at[c, pl.ds(base, ROWS_PER_SUB)])

    return k(nbr_feat, idx1d)


def _self_body(self_ref, w_ref, o_ref):
    o_ref[...] = jnp.dot(self_ref[...], w_ref[...],
                         preferred_element_type=jnp.float32)


def _self_matmul(self_feat, W):
    B = 1024
    return pl.pallas_call(
        _self_body,
        grid=(NPAD // B,),
        in_specs=[
            pl.BlockSpec((B, D), lambda i: (i, 0)),
            pl.BlockSpec((D, D), lambda i: (0, 0)),
        ],
        out_specs=pl.BlockSpec((B, D), lambda i: (i, 0)),
        out_shape=jax.ShapeDtypeStruct((N_NODES, D), jnp.float32),
    )(self_feat, W)


def _tc_body(base_ref, psum_ref, pcnt_ref, w_ref, o_ref):
    i = pl.program_id(0)
    p = psum_ref[0] + psum_ref[1]
    cnt = (pcnt_ref[0, pl.ds(i * 512, 512)]
           + pcnt_ref[1, pl.ds(i * 512, 512)])
    agg = p * (1.0 / jnp.maximum(cnt, 1.0))[:, None]
    o_ref[...] = base_ref[...] + jnp.dot(agg, w_ref[...],
                                         preferred_element_type=jnp.float32)


def _tc_epilogue(base, psum, pcnt, W):
    B = 512
    grid = (NPAD // B,)
    return pl.pallas_call(
        _tc_body,
        grid=grid,
        in_specs=[
            pl.BlockSpec((B, D), lambda i: (i, 0)),
            pl.BlockSpec((NC, B, D), lambda i: (0, i, 0)),
            pl.BlockSpec((NC, NPAD), lambda i: (0, 0)),
            pl.BlockSpec((D, D), lambda i: (1, 0)),
        ],
        out_specs=pl.BlockSpec((B, D), lambda i: (i, 0)),
        out_shape=jax.ShapeDtypeStruct((N_NODES, D), jnp.float32),
    )(base, psum, pcnt, W)


def kernel(self_feat, nbr_feat, relation_src_indices, W):
    idx1d = relation_src_indices.astype(jnp.int32)
    psum, pcnt = _sc_segment_sum(nbr_feat, idx1d)
    base = _self_matmul(self_feat, W)
    out = _tc_epilogue(base, psum, pcnt, W)
    return out


# speedup vs baseline: 1.0603x; 1.0603x over previous
import functools

import jax
import jax.numpy as jnp
from jax import lax
from jax.experimental import pallas as pl
from jax.experimental.pallas import tpu as pltpu
from jax.experimental.pallas import tpu_sc as plsc

N_NODES = 10000
N_EDGES = 320000
D = 128
G = 128
NG = N_EDGES // G
NC = 2
NS = 16
GROUPS_PER_CORE = NG // NC
GROUPS_PER_SUB = -(-GROUPS_PER_CORE // NS)
NPAD = 10240
ROWS_PER_SUB = NPAD // NS


def _sc_segment_sum(nbr_feat, idx1d):
    mesh = plsc.VectorSubcoreMesh(core_axis_name="c", subcore_axis_name="s")

    @functools.partial(
        pl.kernel,
        out_type=(
            jax.ShapeDtypeStruct((NC, NPAD, D), jnp.float32),
            jax.ShapeDtypeStruct((NC, NPAD), jnp.float32),
        ),
        mesh=mesh,
        scratch_types=[
            pltpu.VMEM((2, G), jnp.int32),
            pltpu.VMEM((2, G, D), jnp.float32),
            pltpu.VMEM((ROWS_PER_SUB,), jnp.float32),
            pltpu.VMEM((G,), jnp.float32),
            pltpu.VMEM_SHARED((NPAD, D), jnp.float32),
            pltpu.VMEM_SHARED((NPAD,), jnp.float32),
            pltpu.SemaphoreType.DMA,
            pltpu.SemaphoreType.DMA,
            pltpu.SemaphoreType.DMA,
            pltpu.SemaphoreType.DMA,
        ],
    )
    def k(nbr_hbm, idx_hbm, psum_hbm, pcnt_hbm,
          idx_v, row_v, zc_v, ones_v, acc_sh, cnt_sh, sem0, sem1,
          rsem, csem):
        c = lax.axis_index("c")
        s = lax.axis_index("s")
        zeros16 = jnp.zeros((16,), jnp.float32)
        ones16 = jnp.full((16,), 1.0, jnp.float32)
        sems = (sem0, sem1)

        def z_row(r, carry):
            for kk in range(D // 16):
                row_v[0, r, pl.ds(kk * 16, 16)] = zeros16
            return carry
        lax.fori_loop(0, G, z_row, None)

        def z_cnt(i, carry):
            zc_v[pl.ds(i * 16, 16)] = zeros16
            return carry
        lax.fori_loop(0, ROWS_PER_SUB // 16, z_cnt, None)

        for kk in range(G // 16):
            ones_v[pl.ds(kk * 16, 16)] = ones16

        base = s * ROWS_PER_SUB
        for j in range(ROWS_PER_SUB // G):
            pltpu.sync_copy(row_v.at[0], acc_sh.at[pl.ds(base + j * G, G)])
        pltpu.sync_copy(zc_v, cnt_sh.at[pl.ds(base, ROWS_PER_SUB)])
        plsc.subcore_barrier()

        def pred(t):
            return (t < GROUPS_PER_SUB) & (s * GROUPS_PER_SUB + t < GROUPS_PER_CORE)

        def start(t, b):
            @pl.when(pred(t))
            def _():
                gid = c * GROUPS_PER_CORE + s * GROUPS_PER_SUB + t
                pltpu.async_copy(idx_hbm.at[pl.ds(gid * G, G)], idx_v.at[b],
                                 sems[b])
                pltpu.async_copy(nbr_hbm.at[pl.ds(gid * G, G)], row_v.at[b],
                                 sems[b])

        def proc(t, b):
            @pl.when(pred(t))
            def _():
                pltpu.make_async_copy(idx_hbm.at[pl.ds(0, G)], idx_v.at[b],
                                      sems[b]).wait()
                pltpu.make_async_copy(nbr_hbm.at[pl.ds(0, G)], row_v.at[b],
                                      sems[b]).wait()
                d1 = pltpu.async_copy(row_v.at[b], acc_sh.at[idx_v.at[b]],
                                      rsem, add=True)
                d2 = pltpu.async_copy(ones_v, cnt_sh.at[idx_v.at[b]],
                                      csem, add=True)
                d1.wait()
                d2.wait()

        start(0, 0)

        def pair(p, carry):
            t0 = 2 * p
            start(t0 + 1, 1)
            proc(t0, 0)
            start(t0 + 2, 0)
            proc(t0 + 1, 1)
            return carry
        lax.fori_loop(0, (GROUPS_PER_SUB + 1) // 2, pair, None)
        plsc.subcore_barrier()

        pltpu.sync_copy(acc_sh.at[pl.ds(base, ROWS_PER_SUB)],
                        psum_hbm.at[c, pl.ds(base, ROWS_PER_SUB)])
        pltpu.sync_copy(cnt_sh.at[pl.ds(base, ROWS_PER_SUB)],
                        pcnt_hbm.at[c, pl.ds(base, ROWS_PER_SUB)])

    return k(nbr_feat, idx1d)


def _self_body(self_ref, w_ref, o_ref):
    o_ref[...] = jnp.dot(self_ref[...], w_ref[...],
                         preferred_element_type=jnp.float32)


def _self_matmul(self_feat, W):
    B = 1024
    return pl.pallas_call(
        _self_body,
        grid=(NPAD // B,),
        in_specs=[
            pl.BlockSpec((B, D), lambda i: (i, 0)),
            pl.BlockSpec((D, D), lambda i: (0, 0)),
        ],
        out_specs=pl.BlockSpec((B, D), lambda i: (i, 0)),
        out_shape=jax.ShapeDtypeStruct((N_NODES, D), jnp.float32),
    )(self_feat, W)


def _tc_body(base_ref, psum_ref, pcnt_ref, w_ref, o_ref):
    i = pl.program_id(0)
    p = psum_ref[0] + psum_ref[1]
    cnt = (pcnt_ref[0, pl.ds(i * 1024, 1024)]
           + pcnt_ref[1, pl.ds(i * 1024, 1024)])
    agg = p * (1.0 / jnp.maximum(cnt, 1.0))[:, None]
    o_ref[...] = base_ref[...] + jnp.dot(agg, w_ref[...],
                                         preferred_element_type=jnp.float32)


def _tc_epilogue(base, psum, pcnt, W):
    B = 1024
    grid = (NPAD // B,)
    return pl.pallas_call(
        _tc_body,
        grid=grid,
        in_specs=[
            pl.BlockSpec((B, D), lambda i: (i, 0)),
            pl.BlockSpec((NC, B, D), lambda i: (0, i, 0)),
            pl.BlockSpec((NC, NPAD), lambda i: (0, 0)),
            pl.BlockSpec((D, D), lambda i: (1, 0)),
        ],
        out_specs=pl.BlockSpec((B, D), lambda i: (i, 0)),
        out_shape=jax.ShapeDtypeStruct((N_NODES, D), jnp.float32),
    )(base, psum, pcnt, W)


def kernel(self_feat, nbr_feat, relation_src_indices, W):
    idx1d = relation_src_indices.astype(jnp.int32)
    psum, pcnt = _sc_segment_sum(nbr_feat, idx1d)
    base = _self_matmul(self_feat, W)
    out = _tc_epilogue(base, psum, pcnt, W)
    return out


# speedup vs baseline: 1.1023x; 1.0396x over previous
import functools

import jax
import jax.numpy as jnp
from jax import lax
from jax.experimental import pallas as pl
from jax.experimental.pallas import tpu as pltpu
from jax.experimental.pallas import tpu_sc as plsc

N_NODES = 10000
N_EDGES = 320000
D = 128
G = 80
NG = N_EDGES // G
NC = 2
NS = 16
GROUPS_PER_CORE = NG // NC
GROUPS_PER_SUB = -(-GROUPS_PER_CORE // NS)
NPAD = 10240
ROWS_PER_SUB = NPAD // NS


def _sc_segment_sum(nbr_feat, idx1d):
    mesh = plsc.VectorSubcoreMesh(core_axis_name="c", subcore_axis_name="s")

    @functools.partial(
        pl.kernel,
        out_type=(
            jax.ShapeDtypeStruct((NC, NPAD, D), jnp.float32),
            jax.ShapeDtypeStruct((NC, NPAD), jnp.float32),
        ),
        mesh=mesh,
        scratch_types=[
            pltpu.VMEM((3, G), jnp.int32),
            pltpu.VMEM((3, G, D), jnp.float32),
            pltpu.VMEM((ROWS_PER_SUB,), jnp.float32),
            pltpu.VMEM((G,), jnp.float32),
            pltpu.VMEM_SHARED((NPAD, D), jnp.float32),
            pltpu.VMEM_SHARED((NPAD,), jnp.float32),
            pltpu.SemaphoreType.DMA,
            pltpu.SemaphoreType.DMA,
            pltpu.SemaphoreType.DMA,
            pltpu.SemaphoreType.DMA,
            pltpu.SemaphoreType.DMA,
            pltpu.SemaphoreType.DMA,
        ],
    )
    def k(nbr_hbm, idx_hbm, psum_hbm, pcnt_hbm,
          idx_v, row_v, zc_v, ones_v, acc_sh, cnt_sh,
          dsem0, dsem1, dsem2, ssem0, ssem1, ssem2):
        c = lax.axis_index("c")
        s = lax.axis_index("s")
        zeros16 = jnp.zeros((16,), jnp.float32)
        ones16 = jnp.full((16,), 1.0, jnp.float32)
        sems_d = (dsem0, dsem1, dsem2)
        sems_s = (ssem0, ssem1, ssem2)

        def z_row(r, carry):
            for kk in range(D // 16):
                row_v[0, r, pl.ds(kk * 16, 16)] = zeros16
            return carry
        lax.fori_loop(0, G, z_row, None)

        def z_cnt(i, carry):
            zc_v[pl.ds(i * 16, 16)] = zeros16
            return carry
        lax.fori_loop(0, ROWS_PER_SUB // 16, z_cnt, None)

        for kk in range(G // 16):
            ones_v[pl.ds(kk * 16, 16)] = ones16

        base = s * ROWS_PER_SUB
        for j in range(ROWS_PER_SUB // G):
            pltpu.sync_copy(row_v.at[0], acc_sh.at[pl.ds(base + j * G, G)])
        pltpu.sync_copy(zc_v, cnt_sh.at[pl.ds(base, ROWS_PER_SUB)])
        plsc.subcore_barrier()

        def pred(t):
            return ((t < GROUPS_PER_SUB)
                    & (s * GROUPS_PER_SUB + t < GROUPS_PER_CORE))

        def start(t, b):
            @pl.when(pred(t))
            def _():
                gid = c * GROUPS_PER_CORE + s * GROUPS_PER_SUB + t
                pltpu.async_copy(idx_hbm.at[pl.ds(gid * G, G)], idx_v.at[b],
                                 sems_d[b])
                pltpu.async_copy(nbr_hbm.at[pl.ds(gid * G, G)], row_v.at[b],
                                 sems_d[b])

        def issue(t, b):
            @pl.when(pred(t))
            def _():
                pltpu.make_async_copy(idx_hbm.at[pl.ds(0, G)], idx_v.at[b],
                                      sems_d[b]).wait()
                pltpu.make_async_copy(nbr_hbm.at[pl.ds(0, G)], row_v.at[b],
                                      sems_d[b]).wait()
                pltpu.async_copy(row_v.at[b], acc_sh.at[idx_v.at[b]],
                                 sems_s[b], add=True)
                pltpu.async_copy(ones_v, cnt_sh.at[idx_v.at[b]],
                                 sems_s[b], add=True)

        def wait_scat(t, b):
            @pl.when((t >= 0) & pred(t))
            def _():
                pltpu.make_async_copy(row_v.at[b], acc_sh.at[idx_v.at[b]],
                                      sems_s[b]).wait()
                pltpu.make_async_copy(ones_v, cnt_sh.at[idx_v.at[b]],
                                      sems_s[b]).wait()

        start(0, 0)
        start(1, 1)

        def superstep(q, carry):
            t0 = 3 * q
            for j in range(3):
                t = t0 + j
                bn = (j + 2) % 3
                issue(t, j)
                wait_scat(t - 1, bn)
                start(t + 2, bn)
            return carry
        lax.fori_loop(0, (GROUPS_PER_SUB + 4) // 3, superstep, None)
        plsc.subcore_barrier()

        pltpu.async_copy(acc_sh.at[pl.ds(base, ROWS_PER_SUB)],
                         psum_hbm.at[c, pl.ds(base, ROWS_PER_SUB)], dsem0)
        pltpu.async_copy(cnt_sh.at[pl.ds(base, ROWS_PER_SUB)],
                         pcnt_hbm.at[c, pl.ds(base, ROWS_PER_SUB)], dsem1)
        pltpu.make_async_copy(acc_sh.at[pl.ds(base, ROWS_PER_SUB)],
                              psum_hbm.at[c, pl.ds(base, ROWS_PER_SUB)],
                              dsem0).wait()
        pltpu.make_async_copy(cnt_sh.at[pl.ds(base, ROWS_PER_SUB)],
                              pcnt_hbm.at[c, pl.ds(base, ROWS_PER_SUB)],
                              dsem1).wait()

    return k(nbr_feat, idx1d)


def _self_body(self_ref, w_ref, o_ref):
    o_ref[...] = jnp.dot(self_ref[...], w_ref[...],
                         preferred_element_type=jnp.float32)


def _self_matmul(self_feat, W):
    B = 1024
    return pl.pallas_call(
        _self_body,
        grid=(NPAD // B,),
        in_specs=[
            pl.BlockSpec((B, D), lambda i: (i, 0)),
            pl.BlockSpec((D, D), lambda i: (0, 0)),
        ],
        out_specs=pl.BlockSpec((B, D), lambda i: (i, 0)),
        out_shape=jax.ShapeDtypeStruct((N_NODES, D), jnp.float32),
    )(self_feat, W)


def _tc_body(base_ref, psum_ref, pcnt_ref, w_ref, o_ref):
    i = pl.program_id(0)
    p = psum_ref[0] + psum_ref[1]
    cnt = (pcnt_ref[0, pl.ds(i * 1024, 1024)]
           + pcnt_ref[1, pl.ds(i * 1024, 1024)])
    agg = p * (1.0 / jnp.maximum(cnt, 1.0))[:, None]
    o_ref[...] = base_ref[...] + jnp.dot(agg, w_ref[...],
                                         preferred_element_type=jnp.float32)


def _tc_epilogue(base, psum, pcnt, W):
    B = 1024
    grid = (NPAD // B,)
    return pl.pallas_call(
        _tc_body,
        grid=grid,
        in_specs=[
            pl.BlockSpec((B, D), lambda i: (i, 0)),
            pl.BlockSpec((NC, B, D), lambda i: (0, i, 0)),
            pl.BlockSpec((NC, NPAD), lambda i: (0, 0)),
            pl.BlockSpec((D, D), lambda i: (1, 0)),
        ],
        out_specs=pl.BlockSpec((B, D), lambda i: (i, 0)),
        out_shape=jax.ShapeDtypeStruct((N_NODES, D), jnp.float32),
    )(base, psum, pcnt, W)


def kernel(self_feat, nbr_feat, relation_src_indices, W):
    idx1d = relation_src_indices.astype(jnp.int32)
    psum, pcnt = _sc_segment_sum(nbr_feat, idx1d)
    base = _self_matmul(self_feat, W)
    out = _tc_epilogue(base, psum, pcnt, W)
    return out
